# Initial kernel scaffold; baseline (speedup 1.0000x reference)
import functools

import jax
import jax.numpy as jnp
from jax import lax
from jax.experimental import pallas as pl
from jax.experimental.pallas import tpu as pltpu
from jax.experimental.pallas import tpu_sc as plsc

D = 32
LANES = 16
NSEG = 5
CB = 8


def _rsqrt(x):
    bits = lax.bitcast_convert_type(x, jnp.int32)
    y = lax.bitcast_convert_type(jnp.int32(0x5F3759DF) - (bits >> 1), jnp.float32)
    for _ in range(3):
        y = y * (1.5 - 0.5 * x * y * y)
    return y


@functools.lru_cache(maxsize=None)
def _build(B, L):
    info = plsc.get_sparse_core_info()
    NC, NS = info.num_cores, info.num_subcores
    NW = NC * NS
    assert B % (NW * CB) == 0
    bpw = B // NW
    nchunks = bpw // CB
    F = NSEG * D
    nv = F // LANES

    mesh = plsc.VectorSubcoreMesh(core_axis_name="c", subcore_axis_name="s")

    @functools.partial(
        pl.kernel,
        out_type=jax.ShapeDtypeStruct((B, L, F), jnp.float32),
        mesh=mesh,
        scratch_types=dict(
            useridx=pltpu.VMEM((CB,), jnp.int32),
            posidx=pltpu.VMEM((CB, L), jnp.int32),
            timeidx=pltpu.VMEM((CB, L), jnp.int32),
            user_buf=pltpu.VMEM((CB, D), jnp.float32),
            doc_buf=pltpu.VMEM((CB, D), jnp.float32),
            pos_buf=pltpu.VMEM((2, L, D), jnp.float32),
            road_buf=pltpu.VMEM((2, L, D), jnp.float32),
            time_buf=pltpu.VMEM((2, L, D), jnp.float32),
            stage=pltpu.VMEM((2, L, F), jnp.float32),
            gamma_v=pltpu.VMEM((F,), jnp.float32),
            beta_v=pltpu.VMEM((F,), jnp.float32),
            sem_u=pltpu.SemaphoreType.DMA,
            sem_g0=pltpu.SemaphoreType.DMA,
            sem_g1=pltpu.SemaphoreType.DMA,
            sem_o0=pltpu.SemaphoreType.DMA,
            sem_o1=pltpu.SemaphoreType.DMA,
        ),
    )
    def embed_kernel(user_h, pos_h, time_h, user_W, pos_W, time_W, doc_W,
                     road_W, gamma_h, beta_h, out_h, *, useridx, posidx,
                     timeidx, user_buf, doc_buf, pos_buf, road_buf, time_buf,
                     stage, gamma_v, beta_v, sem_u, sem_g0, sem_g1, sem_o0,
                     sem_o1):
        wid = lax.axis_index("s") * NC + lax.axis_index("c")
        base_b = wid * bpw

        pltpu.sync_copy(gamma_h, gamma_v)
        pltpu.sync_copy(beta_h, beta_v)
        gsegs = [gamma_v[pl.ds(j * LANES, LANES)] for j in range(nv)]
        bsegs = [beta_v[pl.ds(j * LANES, LANES)] for j in range(nv)]
        sem_g = (sem_g0, sem_g1)
        sem_o = (sem_o0, sem_o1)

        def fire_gathers(bi, ph):
            hp = pltpu.async_copy(pos_W.at[posidx.at[bi]], pos_buf.at[ph],
                                  sem_g[ph])
            hr = pltpu.async_copy(road_W.at[posidx.at[bi]], road_buf.at[ph],
                                  sem_g[ph])
            ht = pltpu.async_copy(time_W.at[timeidx.at[bi]], time_buf.at[ph],
                                  sem_g[ph])
            return (hp, hr, ht)

        def chunk_body(c, carry):
            b0 = base_b + c * CB
            pltpu.sync_copy(user_h.at[pl.ds(b0, CB)], useridx)
            pltpu.sync_copy(pos_h.at[pl.ds(b0, CB), :], posidx)
            pltpu.sync_copy(time_h.at[pl.ds(b0, CB), :], timeidx)
            hu = pltpu.async_copy(user_W.at[useridx], user_buf, sem_u)
            hd = pltpu.async_copy(doc_W.at[useridx], doc_buf, sem_u)
            hu.wait()
            hd.wait()

            gh = fire_gathers(0, 0)
            houts = [None] * CB
            for bi in range(CB):
                ph = bi % 2
                nxt = fire_gathers(bi + 1, 1 - ph) if bi + 1 < CB else None
                for h in gh:
                    h.wait()
                gh = nxt
                if bi >= 2:
                    houts[bi - 2].wait()

                u0 = user_buf[bi, pl.ds(0, LANES)]
                u1 = user_buf[bi, pl.ds(LANES, LANES)]
                d0 = doc_buf[bi, pl.ds(0, LANES)]
                d1 = doc_buf[bi, pl.ds(LANES, LANES)]
                us = u0 + u1 + d0 + d1
                usq = u0 * u0 + u1 * u1 + d0 * d0 + d1 * d1

                def row_body(l, _, ph=ph, u0=u0, u1=u1, d0=d0, d1=d1,
                             us=us, usq=usq):
                    t0 = time_buf[ph, l, pl.ds(0, LANES)]
                    t1 = time_buf[ph, l, pl.ds(LANES, LANES)]
                    p0 = pos_buf[ph, l, pl.ds(0, LANES)]
                    p1 = pos_buf[ph, l, pl.ds(LANES, LANES)]
                    r0 = road_buf[ph, l, pl.ds(0, LANES)]
                    r1 = road_buf[ph, l, pl.ds(LANES, LANES)]
                    svec = us + t0 + t1 + p0 + p1 + r0 + r1
                    qvec = (usq + t0 * t0 + t1 * t1 + p0 * p0 + p1 * p1
                            + r0 * r0 + r1 * r1)
                    s = jnp.sum(svec)
                    q = jnp.sum(qvec)
                    mean = s * (1.0 / F)
                    var = q * (1.0 / F) - mean * mean
                    a = _rsqrt(var + 1e-5)
                    off = -mean * a
                    segs = (u0, u1, t0, t1, p0, p1, d0, d1, r0, r1)
                    for j, v in enumerate(segs):
                        stage[ph, l, pl.ds(j * LANES, LANES)] = (
                            (v * a + off) * gsegs[j] + bsegs[j])
                    return 0

                lax.fori_loop(0, L, row_body, 0)
                houts[bi] = pltpu.async_copy(stage.at[ph], out_h.at[b0 + bi],
                                             sem_o[ph])
            houts[CB - 2].wait()
            houts[CB - 1].wait()
            return carry

        lax.fori_loop(0, nchunks, chunk_body, 0)

    return embed_kernel


def kernel(user, pos, time, user_W, pos_W, time_W, doc2vec_W, road_W,
           transport_W, gamma, beta):
    del transport_W
    B, L = pos.shape
    fn = _build(B, L)
    return fn(user.astype(jnp.int32), pos.astype(jnp.int32),
              time.astype(jnp.int32), user_W, pos_W, time_W, doc2vec_W,
              road_W, gamma, beta)


# speedup vs baseline: 4.1688x; 4.1688x over previous
import functools

import jax
import jax.numpy as jnp
from jax import lax
from jax.experimental import pallas as pl
from jax.experimental.pallas import tpu as pltpu
from jax.experimental.pallas import tpu_sc as plsc

D = 32
LANES = 16
NSEG = 5
CB = 8


def _rsqrt(x):
    bits = lax.bitcast_convert_type(x, jnp.int32)
    y = lax.bitcast_convert_type(jnp.int32(0x5F3759DF) - (bits >> 1), jnp.float32)
    for _ in range(3):
        y = y * (1.5 - 0.5 * x * y * y)
    return y


@functools.lru_cache(maxsize=None)
def _build(B, L):
    info = plsc.get_sparse_core_info()
    NC, NS = info.num_cores, info.num_subcores
    NW = NC * NS
    assert B % (NW * CB) == 0
    bpw = B // NW
    nchunks = bpw // CB
    F = NSEG * D
    nv = F // LANES

    mesh = plsc.VectorSubcoreMesh(core_axis_name="c", subcore_axis_name="s")

    @functools.partial(
        pl.kernel,
        out_type=jax.ShapeDtypeStruct((B, L, F), jnp.float32),
        mesh=mesh,
        compiler_params=pltpu.CompilerParams(needs_layout_passes=False,
                                             use_tc_tiling_on_sc=False),
        scratch_types=dict(
            useridx=pltpu.VMEM((CB,), jnp.int32),
            posidx=pltpu.VMEM((CB, L), jnp.int32),
            timeidx=pltpu.VMEM((CB, L), jnp.int32),
            user_buf=pltpu.VMEM((CB, D), jnp.float32),
            doc_buf=pltpu.VMEM((CB, D), jnp.float32),
            pos_buf0=pltpu.VMEM((L, D), jnp.float32),
            pos_buf1=pltpu.VMEM((L, D), jnp.float32),
            road_buf0=pltpu.VMEM((L, D), jnp.float32),
            road_buf1=pltpu.VMEM((L, D), jnp.float32),
            time_buf0=pltpu.VMEM((L, D), jnp.float32),
            time_buf1=pltpu.VMEM((L, D), jnp.float32),
            stage0=pltpu.VMEM((L, F), jnp.float32),
            stage1=pltpu.VMEM((L, F), jnp.float32),
            gamma_v=pltpu.VMEM((F,), jnp.float32),
            beta_v=pltpu.VMEM((F,), jnp.float32),
            sem_u=pltpu.SemaphoreType.DMA,
            sem_g0=pltpu.SemaphoreType.DMA,
            sem_g1=pltpu.SemaphoreType.DMA,
            sem_o0=pltpu.SemaphoreType.DMA,
            sem_o1=pltpu.SemaphoreType.DMA,
        ),
    )
    def embed_kernel(user_h, pos_h, time_h, user_W, pos_W, time_W, doc_W,
                     road_W, gamma_h, beta_h, out_h, *, useridx, posidx,
                     timeidx, user_buf, doc_buf, pos_buf0, pos_buf1,
                     road_buf0, road_buf1, time_buf0, time_buf1, stage0,
                     stage1, gamma_v, beta_v, sem_u, sem_g0, sem_g1, sem_o0,
                     sem_o1):
        wid = lax.axis_index("s") * NC + lax.axis_index("c")
        base_b = wid * bpw

        pltpu.sync_copy(gamma_h, gamma_v)
        pltpu.sync_copy(beta_h, beta_v)
        gsegs = [gamma_v[pl.ds(j * LANES, LANES)] for j in range(nv)]
        bsegs = [beta_v[pl.ds(j * LANES, LANES)] for j in range(nv)]
        sem_g = (sem_g0, sem_g1)
        sem_o = (sem_o0, sem_o1)

        pos_bufs = (pos_buf0, pos_buf1)
        road_bufs = (road_buf0, road_buf1)
        time_bufs = (time_buf0, time_buf1)
        stages = (stage0, stage1)

        def fire_gathers(bi, ph):
            hp = pltpu.async_copy(pos_W.at[posidx.at[bi]], pos_bufs[ph],
                                  sem_g[ph])
            hr = pltpu.async_copy(road_W.at[posidx.at[bi]], road_bufs[ph],
                                  sem_g[ph])
            ht = pltpu.async_copy(time_W.at[timeidx.at[bi]], time_bufs[ph],
                                  sem_g[ph])
            return (hp, hr, ht)

        def chunk_body(c, carry):
            b0 = base_b + c * CB
            pltpu.sync_copy(user_h.at[pl.ds(b0, CB)], useridx)
            pltpu.sync_copy(pos_h.at[pl.ds(b0, CB), :], posidx)
            pltpu.sync_copy(time_h.at[pl.ds(b0, CB), :], timeidx)
            hu = pltpu.async_copy(user_W.at[useridx], user_buf, sem_u)
            hd = pltpu.async_copy(doc_W.at[useridx], doc_buf, sem_u)
            hu.wait()
            hd.wait()

            gh = fire_gathers(0, 0)
            houts = [None] * CB
            for bi in range(CB):
                ph = bi % 2
                nxt = fire_gathers(bi + 1, 1 - ph) if bi + 1 < CB else None
                for h in gh:
                    h.wait()
                gh = nxt
                if bi >= 2:
                    houts[bi - 2].wait()

                u0 = user_buf[bi, pl.ds(0, LANES)]
                u1 = user_buf[bi, pl.ds(LANES, LANES)]
                d0 = doc_buf[bi, pl.ds(0, LANES)]
                d1 = doc_buf[bi, pl.ds(LANES, LANES)]
                us = u0 + u1 + d0 + d1
                usq = u0 * u0 + u1 * u1 + d0 * d0 + d1 * d1

                pbuf, rbuf, tbuf = pos_bufs[ph], road_bufs[ph], time_bufs[ph]
                stg = stages[ph]

                def row_body(l, _, u0=u0, u1=u1, d0=d0, d1=d1, us=us,
                             usq=usq, pbuf=pbuf, rbuf=rbuf, tbuf=tbuf,
                             stg=stg):
                    t0 = tbuf[l, pl.ds(0, LANES)]
                    t1 = tbuf[l, pl.ds(LANES, LANES)]
                    p0 = pbuf[l, pl.ds(0, LANES)]
                    p1 = pbuf[l, pl.ds(LANES, LANES)]
                    r0 = rbuf[l, pl.ds(0, LANES)]
                    r1 = rbuf[l, pl.ds(LANES, LANES)]
                    svec = us + t0 + t1 + p0 + p1 + r0 + r1
                    qvec = (usq + t0 * t0 + t1 * t1 + p0 * p0 + p1 * p1
                            + r0 * r0 + r1 * r1)
                    s = jnp.sum(svec)
                    q = jnp.sum(qvec)
                    mean = s * (1.0 / F)
                    var = q * (1.0 / F) - mean * mean
                    a = _rsqrt(var + 1e-5)
                    off = -mean * a
                    segs = (u0, u1, t0, t1, p0, p1, d0, d1, r0, r1)
                    for j, v in enumerate(segs):
                        stg[l, pl.ds(j * LANES, LANES)] = (
                            (v * a + off) * gsegs[j] + bsegs[j])
                    return 0

                lax.fori_loop(0, L, row_body, 0)
                houts[bi] = pltpu.async_copy(stg, out_h.at[b0 + bi],
                                             sem_o[ph])
            houts[CB - 2].wait()
            houts[CB - 1].wait()
            return carry

        lax.fori_loop(0, nchunks, chunk_body, 0)

    return embed_kernel


def kernel(user, pos, time, user_W, pos_W, time_W, doc2vec_W, road_W,
           transport_W, gamma, beta):
    del transport_W
    B, L = pos.shape
    fn = _build(B, L)
    return fn(user.astype(jnp.int32), pos.astype(jnp.int32),
              time.astype(jnp.int32), user_W, pos_W, time_W, doc2vec_W,
              road_W, gamma, beta)
